# baseline (speedup 1.0000x reference)
import functools

import jax
import jax.numpy as jnp
from jax import lax
from jax.experimental import pallas as pl
from jax.experimental.pallas import tpu as pltpu
from jax.experimental.pallas import tpu_sc as plsc

N = 10000
E = 320000
H = 256
F = 128
RADIUS = 3
T = 2

NC = 2
NS = 16
L = 16

NP = 10240
NRP = NP // NS
EP = E // NS
C = 80
PK = 4 * C
NCHUNK = EP // C
ED = E // (NC * NS)

_mesh = plsc.VectorSubcoreMesh(core_axis_name="c", subcore_axis_name="s")
_sc_params = pltpu.CompilerParams(needs_layout_passes=False)


def _f32(*shape):
    return jax.ShapeDtypeStruct(shape, jnp.float32)


def _m8(x):
    return pl.multiple_of(x, 8)


def _edge_pass_body(core, tile, mode, idxpack, edge_in, gtab, acc,
                    idxb, ab, mb, dstvv, isems, ssems, gsems, scsems, osems,
                    h0_out=None, wv=None, attbs=None, attp_out=None):
    kbase = tile * NCHUNK

    def issue_in(g, b):
        pltpu.async_copy(
            idxpack.at[pl.ds(_m8((kbase + g) * PK), PK)], idxb[b], isems[b])
        pltpu.async_copy(
            edge_in.at[core, pl.ds(_m8(tile * EP + g * C), C)], ab[b], ssems[b])

    def wait_idx(b):
        pltpu.make_async_copy(
            idxpack.at[pl.ds(_m8(kbase * PK), PK)], idxb[b], isems[b]).wait()

    def issue_gather(b):
        pltpu.async_copy(
            gtab.at[idxb[b].at[pl.ds(_m8(core * C), C)]], ab[b], gsems[b],
            add=True)

    def wait_gather(b):
        pltpu.make_async_copy(
            gtab.at[idxb[b].at[pl.ds(_m8(core * C), C)]], ab[b], gsems[b]).wait()

    def wait_stream(b):
        pltpu.make_async_copy(
            edge_in.at[core, pl.ds(_m8(tile * EP), C)], ab[b], ssems[b]).wait()

    def issue_scatter(b):
        pltpu.async_copy(mb[b], acc.at[dstvv[b]], scsems[b], add=True)

    def wait_scatter(b):
        pltpu.make_async_copy(mb[b], acc.at[dstvv[b]], scsems[b]).wait()

    def issue_h0(g, b):
        pltpu.async_copy(
            ab[b], h0_out.at[core, pl.ds(_m8(tile * EP + g * C), C)], osems[b])

    def wait_h0(b):
        pltpu.make_async_copy(
            ab[b], h0_out.at[core, pl.ds(_m8(tile * EP), C)], osems[b]).wait()

    def issue_att(g, b):
        pltpu.async_copy(
            attbs[b],
            attp_out.at[pl.ds(_m8(core * E + tile * EP + g * C), C)], osems[b])

    def wait_att(b):
        pltpu.make_async_copy(
            attbs[b], attp_out.at[pl.ds(_m8(core * E + tile * EP), C)],
            osems[b]).wait()

    issue_in(0, 0)
    issue_in(1, 1)
    wait_idx(0)
    wait_stream(0)
    issue_gather(0)

    def pair(i, carry):
        go = i * 2
        for b in (0, 1):
            g = go + b
            nxt = 1 - b

            @pl.when(g + 1 < NCHUNK)
            def _():
                wait_idx(nxt)
                wait_stream(nxt)
                issue_gather(nxt)

            wait_gather(b)

            @pl.when(g >= 2)
            def _():
                wait_scatter(b)

            for j in range(C // L):
                dstvv[b][pl.ds(j * L, L)] = idxb[b][pl.ds(2 * C + j * L, L)]

            if mode == 'first':
                def egrp0(j, cy):
                    for k in range(L):
                        e = j * L + k
                        for v in range(F // L):
                            sl = pl.ds(v * L, L)
                            ab[b][e, sl] = jnp.maximum(ab[b][e, sl], 0.0)
                    return cy
                lax.fori_loop(0, C // L, egrp0, 0, unroll=2)
                issue_h0(g, b)

                def egrp1(j, cy):
                    dgrp = plsc.bitcast(
                        idxb[b][pl.ds(3 * C + j * L, L)], jnp.float32)
                    for k in range(L):
                        e = j * L + k
                        dv = dgrp[k]
                        for v in range(F // L):
                            sl = pl.ds(v * L, L)
                            mb[b][e, sl] = ab[b][e, sl] * dv
                    return cy
                lax.fori_loop(0, C // L, egrp1, 0, unroll=2)
            elif mode == 'mid':
                def egrp(j, cy):
                    dgrp = plsc.bitcast(
                        idxb[b][pl.ds(3 * C + j * L, L)], jnp.float32)
                    for k in range(L):
                        e = j * L + k
                        dv = dgrp[k]
                        for v in range(F // L):
                            sl = pl.ds(v * L, L)
                            x = jnp.maximum(ab[b][e, sl], 0.0)
                            mb[b][e, sl] = x * dv
                    return cy
                lax.fori_loop(0, C // L, egrp, 0, unroll=2)
            else:
                lane = lax.iota(jnp.int32, L)

                @pl.when(g >= 2)
                def _():
                    wait_att(b)

                def egrp(j, cy):
                    wvs = [wv[pl.ds(v * L, L)] for v in range(F // L)]
                    accv = jnp.zeros((L,), jnp.float32)
                    for k in range(L):
                        e = j * L + k
                        attv = jnp.zeros((L,), jnp.float32)
                        for v in range(F // L):
                            sl = pl.ds(v * L, L)
                            x = jnp.maximum(ab[b][e, sl], 0.0)
                            mb[b][e, sl] = x
                            attv = attv + x * wvs[v]
                        accv = jnp.where(lane == k, jnp.sum(attv), accv)
                    attbs[b][pl.ds(j * L, L)] = accv
                    return cy
                lax.fori_loop(0, C // L, egrp, 0, unroll=2)
                issue_att(g, b)

            issue_scatter(b)

            if mode == 'first':
                wait_h0(b)

            @pl.when(g + 2 < NCHUNK)
            def _():
                issue_in(g + 2, b)
        return carry

    lax.fori_loop(0, NCHUNK // 2, pair, 0)
    wait_scatter(0)
    wait_scatter(1)
    if mode == 'last':
        wait_att(0)
        wait_att(1)


_EDGE_SCRATCH = [
    pltpu.VMEM((PK,), jnp.int32),
    pltpu.VMEM((PK,), jnp.int32),
    pltpu.VMEM((C, F), jnp.float32),
    pltpu.VMEM((C, F), jnp.float32),
    pltpu.VMEM((C, F), jnp.float32),
    pltpu.VMEM((C, F), jnp.float32),
    pltpu.VMEM((C,), jnp.int32),
    pltpu.VMEM((C,), jnp.int32),
    pltpu.VMEM_SHARED((NP, F), jnp.float32),
] + [pltpu.SemaphoreType.DMA] * 10


@functools.partial(
    pl.kernel,
    out_type=(_f32(NC, E, F), _f32(NC, NP, F)),
    mesh=_mesh,
    compiler_params=_sc_params,
    scratch_types=_EDGE_SCRATCH,
)
def _sc_first(gtab, q, idxpack, zerosn, h0_out, agg_out,
              i0, i1, a0, a1, m0, m1, d0, d1, acc,
              is0, is1, ss0, ss1, gs0, gs1, sc0, sc1, os0, os1):
    c = lax.axis_index("c")
    s = lax.axis_index("s")
    r0 = _m8(s * NRP)
    pltpu.sync_copy(zerosn.at[pl.ds(r0, NRP)], acc.at[pl.ds(r0, NRP)])
    plsc.subcore_barrier()
    _edge_pass_body(c, s, 'first', idxpack, q, gtab, acc,
                    [i0, i1], [a0, a1], [m0, m1], [d0, d1],
                    [is0, is1], [ss0, ss1], [gs0, gs1], [sc0, sc1], [os0, os1],
                    h0_out=h0_out)
    plsc.subcore_barrier()
    pltpu.sync_copy(acc.at[pl.ds(r0, NRP)], agg_out.at[c, pl.ds(r0, NRP)])


@functools.partial(
    pl.kernel,
    out_type=_f32(NC, NP, F),
    mesh=_mesh,
    compiler_params=_sc_params,
    scratch_types=_EDGE_SCRATCH,
)
def _sc_mid(h0, mtab, idxpack, zerosn, agg_out,
            i0, i1, a0, a1, m0, m1, d0, d1, acc,
            is0, is1, ss0, ss1, gs0, gs1, sc0, sc1, os0, os1):
    c = lax.axis_index("c")
    s = lax.axis_index("s")
    r0 = _m8(s * NRP)
    pltpu.sync_copy(zerosn.at[pl.ds(r0, NRP)], acc.at[pl.ds(r0, NRP)])
    plsc.subcore_barrier()
    _edge_pass_body(c, s, 'mid', idxpack, h0, mtab, acc,
                    [i0, i1], [a0, a1], [m0, m1], [d0, d1],
                    [is0, is1], [ss0, ss1], [gs0, gs1], [sc0, sc1], [os0, os1])
    plsc.subcore_barrier()
    pltpu.sync_copy(acc.at[pl.ds(r0, NRP)], agg_out.at[c, pl.ds(r0, NRP)])


@functools.partial(
    pl.kernel,
    out_type=(_f32(NC, NP, F), _f32(NC * E)),
    mesh=_mesh,
    compiler_params=_sc_params,
    scratch_types=_EDGE_SCRATCH + [
        pltpu.VMEM((F,), jnp.float32),
        pltpu.VMEM((C,), jnp.float32),
        pltpu.VMEM((C,), jnp.float32),
    ],
)
def _sc_last(h0, mtab, idxpack, watt, zerosn, agg_out, attp_out,
             i0, i1, a0, a1, m0, m1, d0, d1, acc,
             is0, is1, ss0, ss1, gs0, gs1, sc0, sc1, os0, os1,
             wv, attb0, attb1):
    c = lax.axis_index("c")
    s = lax.axis_index("s")
    r0 = _m8(s * NRP)
    pltpu.sync_copy(zerosn.at[pl.ds(r0, NRP)], acc.at[pl.ds(r0, NRP)])
    pltpu.sync_copy(watt.at[pl.ds(_m8(c * F), F)], wv)
    plsc.subcore_barrier()
    _edge_pass_body(c, s, 'last', idxpack, h0, mtab, acc,
                    [i0, i1], [a0, a1], [m0, m1], [d0, d1],
                    [is0, is1], [ss0, ss1], [gs0, gs1], [sc0, sc1], [os0, os1],
                    wv=wv, attbs=[attb0, attb1], attp_out=attp_out)
    plsc.subcore_barrier()
    pltpu.sync_copy(acc.at[pl.ds(r0, NRP)], agg_out.at[c, pl.ds(r0, NRP)])


@functools.partial(
    pl.kernel,
    out_type=_f32(E),
    mesh=_mesh,
    compiler_params=_sc_params,
    scratch_types=[
        pltpu.VMEM((ED,), jnp.int32),
        pltpu.VMEM((ED,), jnp.int32),
        pltpu.VMEM((N,), jnp.float32),
        pltpu.VMEM((N,), jnp.float32),
        pltpu.VMEM((N,), jnp.float32),
        pltpu.VMEM((ED,), jnp.float32),
    ],
)
def _sc_dist(srce, dste, coordx, coordy, coordz, sq_out,
             srcv, dstv, cxv, cyv, czv, sqb):
    c = lax.axis_index("c")
    s = lax.axis_index("s")
    wid = s * NC + c
    base = _m8(wid * ED)
    pltpu.sync_copy(coordx, cxv)
    pltpu.sync_copy(coordy, cyv)
    pltpu.sync_copy(coordz, czv)
    pltpu.sync_copy(srce.at[pl.ds(base, ED)], srcv)
    pltpu.sync_copy(dste.at[pl.ds(base, ED)], dstv)

    def dgrp(j, cy):
        si = srcv[pl.ds(j * L, L)]
        di = dstv[pl.ds(j * L, L)]
        accv = jnp.zeros((L,), jnp.float32)
        for cv in (cxv, cyv, czv):
            d = plsc.load_gather(cv, [si]) - plsc.load_gather(cv, [di])
            accv = accv + d * d
        sqb[pl.ds(j * L, L)] = accv
        return cy

    lax.fori_loop(0, ED // L, dgrp, 0)
    pltpu.sync_copy(sqb, sq_out.at[pl.ds(base, ED)])


def _node_pre_body(af, wa, ba, win, xh_o, g_o):
    xh = jnp.maximum(af[...] @ wa[...] + ba[...], 0.0)
    xh_o[...] = xh
    g = xh @ win[...]
    g_o[0] = g[:, :F]
    g_o[1] = g[:, F:]


def _node_pre(af, wa, ba, win_n):
    bn = 2048
    return pl.pallas_call(
        _node_pre_body,
        grid=(NP // bn,),
        in_specs=[
            pl.BlockSpec((bn, 70), lambda i: (i, 0)),
            pl.BlockSpec((70, H), lambda i: (0, 0)),
            pl.BlockSpec((1, H), lambda i: (0, 0)),
            pl.BlockSpec((H, H), lambda i: (0, 0)),
        ],
        out_specs=[
            pl.BlockSpec((bn, H), lambda i: (i, 0)),
            pl.BlockSpec((NC, bn, F), lambda i: (0, i, 0)),
        ],
        out_shape=[_f32(N, H), _f32(NC, NP, F)],
    )(af, wa, ba, win_n)


def _edge_pre_body(ef, we, be, win, q_o):
    eh = jnp.maximum(ef[...] @ we[...] + be[...], 0.0)
    qq = eh @ win[...]
    q_o[0] = qq[:, :F]
    q_o[1] = qq[:, F:]


def _edge_pre(ef, we, be, win_e):
    be_blk = 4000
    return pl.pallas_call(
        _edge_pre_body,
        grid=(E // be_blk,),
        in_specs=[
            pl.BlockSpec((be_blk, 14), lambda i: (i, 0)),
            pl.BlockSpec((14, F), lambda i: (0, 0)),
            pl.BlockSpec((1, F), lambda i: (0, 0)),
            pl.BlockSpec((F, H), lambda i: (0, 0)),
        ],
        out_specs=pl.BlockSpec((NC, be_blk, F), lambda i: (0, i, 0)),
        out_shape=_f32(NC, E, F),
    )(ef, we, be, win_e)


def _decay_body(aw, out):
    a = aw[...]
    diss = jnp.where(a == 0.0, jnp.float32(-1.0), jnp.log(a) * 2.0)
    out[...] = jnp.where(a == 1.0, jnp.float32(1.0), diss)


def _decay(aw):
    a2 = aw.reshape(E // 128, 128)
    out = pl.pallas_call(
        _decay_body,
        out_shape=jax.ShapeDtypeStruct(a2.shape, jnp.float32),
    )(a2)
    return out.reshape(E)


def _mm_body(agg, wh, out):
    a = jnp.concatenate([agg[0], agg[1]], axis=1)
    m = a @ wh[...]
    out[0] = m[:, :F]
    out[1] = m[:, F:]


def _mm_agg(agg, wh):
    bn = 2048
    return pl.pallas_call(
        _mm_body,
        grid=(NP // bn,),
        in_specs=[
            pl.BlockSpec((NC, bn, F), lambda i: (0, i, 0)),
            pl.BlockSpec((H, H), lambda i: (0, 0)),
        ],
        out_specs=pl.BlockSpec((NC, bn, F), lambda i: (0, i, 0)),
        out_shape=_f32(NC, NP, F),
    )(agg, wh)


def _hnode_body(xh, ni, wo, out):
    a = jnp.concatenate([xh[...], ni[0], ni[1]], axis=1)
    out[...] = jnp.maximum(a @ wo[...], 0.0)


def _hnode(xh, ni, wo):
    bn = 2000
    return pl.pallas_call(
        _hnode_body,
        grid=(N // bn,),
        in_specs=[
            pl.BlockSpec((bn, H), lambda i: (i, 0)),
            pl.BlockSpec((NC, bn, F), lambda i: (0, i, 0)),
            pl.BlockSpec((2 * H, H), lambda i: (0, 0)),
        ],
        out_specs=pl.BlockSpec((bn, H), lambda i: (i, 0)),
        out_shape=_f32(N, H),
    )(xh, ni, wo)


def _att_body(p, sq, out):
    s = p[0] + p[1] + jnp.sqrt(sq[...] + 1e-12)
    out[...] = jax.nn.sigmoid(s)


def _att(attp, sqdist):
    p = attp.reshape(NC, E // 128, 128)
    sq = sqdist.reshape(E // 128, 128)
    out = pl.pallas_call(
        _att_body,
        out_shape=jax.ShapeDtypeStruct(sq.shape, jnp.float32),
    )(p, sq)
    return out.reshape(E)


def _readout_body(hn_ref, wan, wz, wr, wh, f1, b1, f2, b2, f3, b3, f4, b4,
                  zk_o, a1_o):
    hn = hn_ref[...]
    state = jnp.mean(hn, axis=0, keepdims=True)
    a = None
    for _ in range(T):
        w = wan[...] + state
        s = jnp.sum(hn * w, axis=1, keepdims=True)
        mx = jnp.max(s, axis=0, keepdims=True)
        ex = jnp.exp(s - mx)
        a = ex / jnp.sum(ex, axis=0, keepdims=True)
        ctx = jnp.sum(a * hn, axis=0, keepdims=True)
        zin = jnp.concatenate([ctx, state], axis=1)
        z = jax.nn.sigmoid(zin @ wz[...])
        r = jax.nn.sigmoid(zin @ wr[...])
        cand = jnp.tanh(jnp.concatenate([ctx, r * state], axis=1) @ wh[...])
        state = (1.0 - z) * state + z * cand
    a1_o[...] = a
    zk = jnp.maximum(state @ f1[...] + b1[...], 0.0)
    zk = jnp.maximum(zk @ f2[...] + b2[...], 0.0)
    zk = jnp.maximum(zk @ f3[...] + b3[...], 0.0)
    zk_o[...] = zk @ f4[...] + b4[...]


def _readout(hn, wan, wz, wr, wh, f1, b1, f2, b2, f3, b3, f4, b4):
    return pl.pallas_call(
        _readout_body,
        out_shape=[_f32(1, 1), _f32(N, 1)],
    )(hn, wan, wz, wr, wh, f1, b1, f2, b2, f3, b3, f4, b4)


def kernel(atom_feature, atom_coordinate, edge_feature, attention_weight,
           edge_index, p_or_l, W_atom, b_atom, W_edge, b_edge, W_in, W_h, W_o,
           w_att_e, w_att_n, Wz, Wr, Wh, F1, b1, F2, b2, F3, b3, F4, b4):
    src = edge_index[0].astype(jnp.int32)
    dst = edge_index[1].astype(jnp.int32)
    cx = atom_coordinate[:, 0]
    cy = atom_coordinate[:, 1]
    cz = atom_coordinate[:, 2]
    zerosn = jnp.zeros((NP, F), jnp.float32)

    x_h, g3 = _node_pre(atom_feature, W_atom, b_atom.reshape(1, H), W_in[:H])
    gtab = g3.reshape(NC * NP, F)
    q = _edge_pre(edge_feature, W_edge, b_edge.reshape(1, F), W_in[H:])
    decay = _decay(attention_weight)

    decbits = lax.bitcast_convert_type(decay, jnp.int32)
    idxpack = jnp.stack(
        [src.reshape(-1, C), (src + NP).reshape(-1, C),
         dst.reshape(-1, C), decbits.reshape(-1, C)], axis=1).reshape(-1)

    sqdist = _sc_dist(src, dst, cx, cy, cz)
    h0, agg = _sc_first(gtab, q, idxpack, zerosn)
    for _ in range(RADIUS - 1):
        mtab = _mm_agg(agg, W_h).reshape(NC * NP, F)
        agg = _sc_mid(h0, mtab, idxpack, zerosn)
    mtab = _mm_agg(agg, W_h).reshape(NC * NP, F)
    node_in, attp = _sc_last(h0, mtab, idxpack, w_att_e, zerosn)

    att1 = _att(attp, sqdist)
    hn = _hnode(x_h, node_in, W_o)
    zk, a1 = _readout(hn, w_att_n.reshape(1, H), Wz, Wr, Wh,
                      F1, b1.reshape(1, -1), F2, b2.reshape(1, -1),
                      F3, b3.reshape(1, -1), F4, b4.reshape(1, -1))
    return zk, att1, a1.reshape(N)


# speedup vs baseline: 1.0042x; 1.0042x over previous
import functools

import jax
import jax.numpy as jnp
from jax import lax
from jax.experimental import pallas as pl
from jax.experimental.pallas import tpu as pltpu
from jax.experimental.pallas import tpu_sc as plsc

N = 10000
E = 320000
H = 256
F = 128
RADIUS = 3
T = 2

NC = 2
NS = 16
L = 16

NP = 10240
NRP = NP // NS
EP = E // NS
C = 80
PK = 4 * C
NCHUNK = EP // C
ED = E // (NC * NS)

_mesh = plsc.VectorSubcoreMesh(core_axis_name="c", subcore_axis_name="s")
_sc_params = pltpu.CompilerParams(needs_layout_passes=False)


def _f32(*shape):
    return jax.ShapeDtypeStruct(shape, jnp.float32)


def _m8(x):
    return pl.multiple_of(x, 8)


def _edge_pass_body(core, tile, mode, idxpack, edge_in, gtab, acc,
                    idxb, ab, mb, dstvv, isems, ssems, gsems, scsems, osems,
                    h0_out=None, wv=None, attbs=None, attp_out=None):
    kbase = tile * NCHUNK

    def issue_in(g, b):
        pltpu.async_copy(
            idxpack.at[pl.ds(_m8((kbase + g) * PK), PK)], idxb[b], isems[b])
        pltpu.async_copy(
            edge_in.at[core, pl.ds(_m8(tile * EP + g * C), C)], ab[b], ssems[b])

    def wait_idx(b):
        pltpu.make_async_copy(
            idxpack.at[pl.ds(_m8(kbase * PK), PK)], idxb[b], isems[b]).wait()

    def issue_gather(b):
        pltpu.async_copy(
            gtab.at[idxb[b].at[pl.ds(_m8(core * C), C)]], ab[b], gsems[b],
            add=True)

    def wait_gather(b):
        pltpu.make_async_copy(
            gtab.at[idxb[b].at[pl.ds(_m8(core * C), C)]], ab[b], gsems[b]).wait()

    def wait_stream(b):
        pltpu.make_async_copy(
            edge_in.at[core, pl.ds(_m8(tile * EP), C)], ab[b], ssems[b]).wait()

    def issue_scatter(b):
        pltpu.async_copy(mb[b], acc.at[dstvv[b]], scsems[b], add=True)

    def wait_scatter(b):
        pltpu.make_async_copy(mb[b], acc.at[dstvv[b]], scsems[b]).wait()

    def issue_h0(g, b):
        pltpu.async_copy(
            ab[b], h0_out.at[core, pl.ds(_m8(tile * EP + g * C), C)], osems[b])

    def wait_h0(b):
        pltpu.make_async_copy(
            ab[b], h0_out.at[core, pl.ds(_m8(tile * EP), C)], osems[b]).wait()

    def issue_att(g, b):
        pltpu.async_copy(
            attbs[b],
            attp_out.at[pl.ds(_m8(core * E + tile * EP + g * C), C)], osems[b])

    def wait_att(b):
        pltpu.make_async_copy(
            attbs[b], attp_out.at[pl.ds(_m8(core * E + tile * EP), C)],
            osems[b]).wait()

    issue_in(0, 0)
    issue_in(1, 1)
    wait_idx(0)
    wait_stream(0)
    issue_gather(0)

    def pair(i, carry):
        go = i * 2
        for b in (0, 1):
            g = go + b
            nxt = 1 - b

            @pl.when(g + 1 < NCHUNK)
            def _():
                wait_idx(nxt)
                wait_stream(nxt)
                issue_gather(nxt)

            wait_gather(b)

            @pl.when(g >= 2)
            def _():
                wait_scatter(b)

            for j in range(C // L):
                dstvv[b][pl.ds(j * L, L)] = idxb[b][pl.ds(2 * C + j * L, L)]

            if mode == 'first':
                def egrp0(j, cy):
                    for k in range(L):
                        e = j * L + k
                        for v in range(F // L):
                            sl = pl.ds(v * L, L)
                            ab[b][e, sl] = jnp.maximum(ab[b][e, sl], 0.0)
                    return cy
                lax.fori_loop(0, C // L, egrp0, 0)
                issue_h0(g, b)

                def egrp1(j, cy):
                    dgrp = plsc.bitcast(
                        idxb[b][pl.ds(3 * C + j * L, L)], jnp.float32)
                    for k in range(L):
                        e = j * L + k
                        dv = dgrp[k]
                        for v in range(F // L):
                            sl = pl.ds(v * L, L)
                            mb[b][e, sl] = ab[b][e, sl] * dv
                    return cy
                lax.fori_loop(0, C // L, egrp1, 0)
            elif mode == 'mid':
                def egrp(j, cy):
                    dgrp = plsc.bitcast(
                        idxb[b][pl.ds(3 * C + j * L, L)], jnp.float32)
                    for k in range(L):
                        e = j * L + k
                        dv = dgrp[k]
                        for v in range(F // L):
                            sl = pl.ds(v * L, L)
                            x = jnp.maximum(ab[b][e, sl], 0.0)
                            mb[b][e, sl] = x * dv
                    return cy
                lax.fori_loop(0, C // L, egrp, 0)
            else:
                lane = lax.iota(jnp.int32, L)

                @pl.when(g >= 2)
                def _():
                    wait_att(b)

                def egrp(j, cy):
                    wvs = [wv[pl.ds(v * L, L)] for v in range(F // L)]
                    accv = jnp.zeros((L,), jnp.float32)
                    for k in range(L):
                        e = j * L + k
                        attv = jnp.zeros((L,), jnp.float32)
                        for v in range(F // L):
                            sl = pl.ds(v * L, L)
                            x = jnp.maximum(ab[b][e, sl], 0.0)
                            mb[b][e, sl] = x
                            attv = attv + x * wvs[v]
                        accv = jnp.where(lane == k, jnp.sum(attv), accv)
                    attbs[b][pl.ds(j * L, L)] = accv
                    return cy
                lax.fori_loop(0, C // L, egrp, 0)
                issue_att(g, b)

            issue_scatter(b)

            if mode == 'first':
                wait_h0(b)

            @pl.when(g + 2 < NCHUNK)
            def _():
                issue_in(g + 2, b)
        return carry

    lax.fori_loop(0, NCHUNK // 2, pair, 0)
    wait_scatter(0)
    wait_scatter(1)
    if mode == 'last':
        wait_att(0)
        wait_att(1)


_EDGE_SCRATCH = [
    pltpu.VMEM((PK,), jnp.int32),
    pltpu.VMEM((PK,), jnp.int32),
    pltpu.VMEM((C, F), jnp.float32),
    pltpu.VMEM((C, F), jnp.float32),
    pltpu.VMEM((C, F), jnp.float32),
    pltpu.VMEM((C, F), jnp.float32),
    pltpu.VMEM((C,), jnp.int32),
    pltpu.VMEM((C,), jnp.int32),
    pltpu.VMEM_SHARED((NP, F), jnp.float32),
] + [pltpu.SemaphoreType.DMA] * 10


@functools.partial(
    pl.kernel,
    out_type=(_f32(NC, E, F), _f32(NC, NP, F)),
    mesh=_mesh,
    compiler_params=_sc_params,
    scratch_types=_EDGE_SCRATCH,
)
def _sc_first(gtab, q, idxpack, zerosn, h0_out, agg_out,
              i0, i1, a0, a1, m0, m1, d0, d1, acc,
              is0, is1, ss0, ss1, gs0, gs1, sc0, sc1, os0, os1):
    c = lax.axis_index("c")
    s = lax.axis_index("s")
    r0 = _m8(s * NRP)
    pltpu.sync_copy(zerosn.at[pl.ds(r0, NRP)], acc.at[pl.ds(r0, NRP)])
    plsc.subcore_barrier()
    _edge_pass_body(c, s, 'first', idxpack, q, gtab, acc,
                    [i0, i1], [a0, a1], [m0, m1], [d0, d1],
                    [is0, is1], [ss0, ss1], [gs0, gs1], [sc0, sc1], [os0, os1],
                    h0_out=h0_out)
    plsc.subcore_barrier()
    pltpu.sync_copy(acc.at[pl.ds(r0, NRP)], agg_out.at[c, pl.ds(r0, NRP)])


@functools.partial(
    pl.kernel,
    out_type=_f32(NC, NP, F),
    mesh=_mesh,
    compiler_params=_sc_params,
    scratch_types=_EDGE_SCRATCH,
)
def _sc_mid(h0, mtab, idxpack, zerosn, agg_out,
            i0, i1, a0, a1, m0, m1, d0, d1, acc,
            is0, is1, ss0, ss1, gs0, gs1, sc0, sc1, os0, os1):
    c = lax.axis_index("c")
    s = lax.axis_index("s")
    r0 = _m8(s * NRP)
    pltpu.sync_copy(zerosn.at[pl.ds(r0, NRP)], acc.at[pl.ds(r0, NRP)])
    plsc.subcore_barrier()
    _edge_pass_body(c, s, 'mid', idxpack, h0, mtab, acc,
                    [i0, i1], [a0, a1], [m0, m1], [d0, d1],
                    [is0, is1], [ss0, ss1], [gs0, gs1], [sc0, sc1], [os0, os1])
    plsc.subcore_barrier()
    pltpu.sync_copy(acc.at[pl.ds(r0, NRP)], agg_out.at[c, pl.ds(r0, NRP)])


@functools.partial(
    pl.kernel,
    out_type=(_f32(NC, NP, F), _f32(NC * E)),
    mesh=_mesh,
    compiler_params=_sc_params,
    scratch_types=_EDGE_SCRATCH + [
        pltpu.VMEM((F,), jnp.float32),
        pltpu.VMEM((C,), jnp.float32),
        pltpu.VMEM((C,), jnp.float32),
    ],
)
def _sc_last(h0, mtab, idxpack, watt, zerosn, agg_out, attp_out,
             i0, i1, a0, a1, m0, m1, d0, d1, acc,
             is0, is1, ss0, ss1, gs0, gs1, sc0, sc1, os0, os1,
             wv, attb0, attb1):
    c = lax.axis_index("c")
    s = lax.axis_index("s")
    r0 = _m8(s * NRP)
    pltpu.sync_copy(zerosn.at[pl.ds(r0, NRP)], acc.at[pl.ds(r0, NRP)])
    pltpu.sync_copy(watt.at[pl.ds(_m8(c * F), F)], wv)
    plsc.subcore_barrier()
    _edge_pass_body(c, s, 'last', idxpack, h0, mtab, acc,
                    [i0, i1], [a0, a1], [m0, m1], [d0, d1],
                    [is0, is1], [ss0, ss1], [gs0, gs1], [sc0, sc1], [os0, os1],
                    wv=wv, attbs=[attb0, attb1], attp_out=attp_out)
    plsc.subcore_barrier()
    pltpu.sync_copy(acc.at[pl.ds(r0, NRP)], agg_out.at[c, pl.ds(r0, NRP)])


@functools.partial(
    pl.kernel,
    out_type=_f32(E),
    mesh=_mesh,
    compiler_params=_sc_params,
    scratch_types=[
        pltpu.VMEM((ED,), jnp.int32),
        pltpu.VMEM((ED,), jnp.int32),
        pltpu.VMEM((N,), jnp.float32),
        pltpu.VMEM((N,), jnp.float32),
        pltpu.VMEM((N,), jnp.float32),
        pltpu.VMEM((ED,), jnp.float32),
    ],
)
def _sc_dist(srce, dste, coordx, coordy, coordz, sq_out,
             srcv, dstv, cxv, cyv, czv, sqb):
    c = lax.axis_index("c")
    s = lax.axis_index("s")
    wid = s * NC + c
    base = _m8(wid * ED)
    pltpu.sync_copy(coordx, cxv)
    pltpu.sync_copy(coordy, cyv)
    pltpu.sync_copy(coordz, czv)
    pltpu.sync_copy(srce.at[pl.ds(base, ED)], srcv)
    pltpu.sync_copy(dste.at[pl.ds(base, ED)], dstv)

    def dgrp(j, cy):
        si = srcv[pl.ds(j * L, L)]
        di = dstv[pl.ds(j * L, L)]
        accv = jnp.zeros((L,), jnp.float32)
        for cv in (cxv, cyv, czv):
            d = plsc.load_gather(cv, [si]) - plsc.load_gather(cv, [di])
            accv = accv + d * d
        sqb[pl.ds(j * L, L)] = accv
        return cy

    lax.fori_loop(0, ED // L, dgrp, 0)
    pltpu.sync_copy(sqb, sq_out.at[pl.ds(base, ED)])


def _node_pre_body(af, wa, ba, win, xh_o, g_o):
    xh = jnp.maximum(af[...] @ wa[...] + ba[...], 0.0)
    xh_o[...] = xh
    g = xh @ win[...]
    g_o[0] = g[:, :F]
    g_o[1] = g[:, F:]


def _node_pre(af, wa, ba, win_n):
    bn = 2048
    return pl.pallas_call(
        _node_pre_body,
        grid=(NP // bn,),
        in_specs=[
            pl.BlockSpec((bn, 70), lambda i: (i, 0)),
            pl.BlockSpec((70, H), lambda i: (0, 0)),
            pl.BlockSpec((1, H), lambda i: (0, 0)),
            pl.BlockSpec((H, H), lambda i: (0, 0)),
        ],
        out_specs=[
            pl.BlockSpec((bn, H), lambda i: (i, 0)),
            pl.BlockSpec((NC, bn, F), lambda i: (0, i, 0)),
        ],
        out_shape=[_f32(N, H), _f32(NC, NP, F)],
    )(af, wa, ba, win_n)


def _edge_pre_body(ef, we, be, win, q_o):
    eh = jnp.maximum(ef[...] @ we[...] + be[...], 0.0)
    qq = eh @ win[...]
    q_o[0] = qq[:, :F]
    q_o[1] = qq[:, F:]


def _edge_pre(ef, we, be, win_e):
    be_blk = 4000
    return pl.pallas_call(
        _edge_pre_body,
        grid=(E // be_blk,),
        in_specs=[
            pl.BlockSpec((be_blk, 14), lambda i: (i, 0)),
            pl.BlockSpec((14, F), lambda i: (0, 0)),
            pl.BlockSpec((1, F), lambda i: (0, 0)),
            pl.BlockSpec((F, H), lambda i: (0, 0)),
        ],
        out_specs=pl.BlockSpec((NC, be_blk, F), lambda i: (0, i, 0)),
        out_shape=_f32(NC, E, F),
    )(ef, we, be, win_e)


def _decay_body(aw, out):
    a = aw[...]
    diss = jnp.where(a == 0.0, jnp.float32(-1.0), jnp.log(a) * 2.0)
    out[...] = jnp.where(a == 1.0, jnp.float32(1.0), diss)


def _decay(aw):
    a2 = aw.reshape(E // 128, 128)
    out = pl.pallas_call(
        _decay_body,
        out_shape=jax.ShapeDtypeStruct(a2.shape, jnp.float32),
    )(a2)
    return out.reshape(E)


def _mm_body(agg, wh, out):
    a = jnp.concatenate([agg[0], agg[1]], axis=1)
    m = a @ wh[...]
    out[0] = m[:, :F]
    out[1] = m[:, F:]


def _mm_agg(agg, wh):
    bn = 2048
    return pl.pallas_call(
        _mm_body,
        grid=(NP // bn,),
        in_specs=[
            pl.BlockSpec((NC, bn, F), lambda i: (0, i, 0)),
            pl.BlockSpec((H, H), lambda i: (0, 0)),
        ],
        out_specs=pl.BlockSpec((NC, bn, F), lambda i: (0, i, 0)),
        out_shape=_f32(NC, NP, F),
    )(agg, wh)


def _hnode_body(xh, ni, wo, out):
    a = jnp.concatenate([xh[...], ni[0], ni[1]], axis=1)
    out[...] = jnp.maximum(a @ wo[...], 0.0)


def _hnode(xh, ni, wo):
    bn = 2000
    return pl.pallas_call(
        _hnode_body,
        grid=(N // bn,),
        in_specs=[
            pl.BlockSpec((bn, H), lambda i: (i, 0)),
            pl.BlockSpec((NC, bn, F), lambda i: (0, i, 0)),
            pl.BlockSpec((2 * H, H), lambda i: (0, 0)),
        ],
        out_specs=pl.BlockSpec((bn, H), lambda i: (i, 0)),
        out_shape=_f32(N, H),
    )(xh, ni, wo)


def _att_body(p, sq, out):
    s = p[0] + p[1] + jnp.sqrt(sq[...] + 1e-12)
    out[...] = jax.nn.sigmoid(s)


def _att(attp, sqdist):
    p = attp.reshape(NC, E // 128, 128)
    sq = sqdist.reshape(E // 128, 128)
    out = pl.pallas_call(
        _att_body,
        out_shape=jax.ShapeDtypeStruct(sq.shape, jnp.float32),
    )(p, sq)
    return out.reshape(E)


def _readout_body(hn_ref, wan, wz, wr, wh, f1, b1, f2, b2, f3, b3, f4, b4,
                  zk_o, a1_o):
    hn = hn_ref[...]
    state = jnp.mean(hn, axis=0, keepdims=True)
    a = None
    for _ in range(T):
        s = (jnp.sum(hn * wan[...], axis=1, keepdims=True)
             + jnp.sum(hn * state, axis=1, keepdims=True))
        mx = jnp.max(s, axis=0, keepdims=True)
        ex = jnp.exp(s - mx)
        a = ex / jnp.sum(ex, axis=0, keepdims=True)
        ctx = jnp.sum(a * hn, axis=0, keepdims=True)
        zin = jnp.concatenate([ctx, state], axis=1)
        z = jax.nn.sigmoid(zin @ wz[...])
        r = jax.nn.sigmoid(zin @ wr[...])
        cand = jnp.tanh(jnp.concatenate([ctx, r * state], axis=1) @ wh[...])
        state = (1.0 - z) * state + z * cand
    a1_o[...] = a
    zk = jnp.maximum(state @ f1[...] + b1[...], 0.0)
    zk = jnp.maximum(zk @ f2[...] + b2[...], 0.0)
    zk = jnp.maximum(zk @ f3[...] + b3[...], 0.0)
    zk_o[...] = zk @ f4[...] + b4[...]


def _readout(hn, wan, wz, wr, wh, f1, b1, f2, b2, f3, b3, f4, b4):
    return pl.pallas_call(
        _readout_body,
        out_shape=[_f32(1, 1), _f32(N, 1)],
    )(hn, wan, wz, wr, wh, f1, b1, f2, b2, f3, b3, f4, b4)


def kernel(atom_feature, atom_coordinate, edge_feature, attention_weight,
           edge_index, p_or_l, W_atom, b_atom, W_edge, b_edge, W_in, W_h, W_o,
           w_att_e, w_att_n, Wz, Wr, Wh, F1, b1, F2, b2, F3, b3, F4, b4):
    src = edge_index[0].astype(jnp.int32)
    dst = edge_index[1].astype(jnp.int32)
    cx = atom_coordinate[:, 0]
    cy = atom_coordinate[:, 1]
    cz = atom_coordinate[:, 2]
    zerosn = jnp.zeros((NP, F), jnp.float32)

    x_h, g3 = _node_pre(atom_feature, W_atom, b_atom.reshape(1, H), W_in[:H])
    gtab = g3.reshape(NC * NP, F)
    q = _edge_pre(edge_feature, W_edge, b_edge.reshape(1, F), W_in[H:])
    decay = _decay(attention_weight)

    decbits = lax.bitcast_convert_type(decay, jnp.int32)
    idxpack = jnp.stack(
        [src.reshape(-1, C), (src + NP).reshape(-1, C),
         dst.reshape(-1, C), decbits.reshape(-1, C)], axis=1).reshape(-1)

    sqdist = _sc_dist(src, dst, cx, cy, cz)
    h0, agg = _sc_first(gtab, q, idxpack, zerosn)
    for _ in range(RADIUS - 1):
        mtab = _mm_agg(agg, W_h).reshape(NC * NP, F)
        agg = _sc_mid(h0, mtab, idxpack, zerosn)
    mtab = _mm_agg(agg, W_h).reshape(NC * NP, F)
    node_in, attp = _sc_last(h0, mtab, idxpack, w_att_e, zerosn)

    att1 = _att(attp, sqdist)
    hn = _hnode(x_h, node_in, W_o)
    zk, a1 = _readout(hn, w_att_n.reshape(1, H), Wz, Wr, Wh,
                      F1, b1.reshape(1, -1), F2, b2.reshape(1, -1),
                      F3, b3.reshape(1, -1), F4, b4.reshape(1, -1))
    return zk, att1, a1.reshape(N)
